# Initial kernel scaffold; baseline (speedup 1.0000x reference)
import jax
import jax.numpy as jnp
from jax.experimental import pallas as pl


def kernel(x):
    raise NotImplementedError("write your pallas kernel here")


# speedup vs baseline: 1.9278x; 1.9278x over previous
import jax
import jax.numpy as jnp
from jax.experimental import pallas as pl

_NUM_CLASSES = 1000
_BLOCK = 512


def _body(x_ref, o_ref):
    x = x_ref[...]
    sel = x[:, 0:1].astype(jnp.int32)
    cols = jax.lax.broadcasted_iota(jnp.int32, (x.shape[0], _NUM_CLASSES), 1)
    oh = jnp.where(cols == sel, 1.0, 0.0).astype(x.dtype)
    o_ref[...] = jnp.concatenate([oh, x], axis=1)


def kernel(x):
    batch, feat = x.shape
    out_cols = _NUM_CLASSES + feat
    return pl.pallas_call(
        _body,
        grid=(batch // _BLOCK,),
        in_specs=[pl.BlockSpec((_BLOCK, feat), lambda i: (i, 0))],
        out_specs=pl.BlockSpec((_BLOCK, out_cols), lambda i: (i, 0)),
        out_shape=jax.ShapeDtypeStruct((batch, out_cols), x.dtype),
    )(x)
